# baseline (speedup 1.0000x reference)
import functools

import jax
import jax.numpy as jnp
from jax import lax
from jax.experimental import pallas as pl
from jax.experimental.pallas import tpu as pltpu
from jax.experimental.pallas import tpu_sc as plsc

N = 10000
E = 320000
D = 16
NT = 16
NP = 10240
NPT = NP // NT
CW = 128
ROWS = 2560
EP = ROWS * CW
RPT = ROWS // NT
GS = 4
NG = RPT // GS

_MESH = plsc.VectorSubcoreMesh(core_axis_name="c", subcore_axis_name="s",
                               num_cores=1)


def _zero_rows(ref, n):
  z = jnp.zeros((D,), jnp.float32)

  def body(i, c):
    ref[i, :] = z
    return c

  lax.fori_loop(0, n, body, 0)


@functools.partial(
    pl.kernel,
    out_type=(
        jax.ShapeDtypeStruct((NP, D), jnp.float32),
        jax.ShapeDtypeStruct((NP, D), jnp.float32),
        jax.ShapeDtypeStruct((NP, D), jnp.float32),
    ),
    mesh=_MESH,
    scratch_types=[
        pltpu.VMEM((NP,), jnp.float32),
        pltpu.VMEM((NT, NPT), jnp.float32),
        pltpu.VMEM((NPT,), jnp.float32),
        pltpu.VMEM((NPT, D), jnp.float32),
        pltpu.VMEM((NPT, D), jnp.float32),
        pltpu.VMEM((RPT, CW), jnp.int32),
        pltpu.VMEM((RPT, CW), jnp.int32),
        pltpu.VMEM((2, GS, CW, D), jnp.float32),
        pltpu.VMEM((D,), jnp.float32),
        pltpu.SemaphoreType.DMA,
        pltpu.SemaphoreType.DMA,
        pltpu.SemaphoreType.DMA,
        pltpu.VMEM_SHARED((NT, NP), jnp.float32),
        pltpu.VMEM_SHARED((NP, D), jnp.float32),
        pltpu.VMEM_SHARED((NP, D), jnp.float32),
    ],
    compiler_params=pltpu.CompilerParams(needs_layout_passes=False, use_tc_tiling_on_sc=False),
)
def _prop_kernel(h1_hbm, src_hbm, dst_hbm, b1_hbm,
                 p2_hbm, g1_hbm, g2_hbm,
                 deg_v, tmp_v, dinv_v, g_v, s_v, si_v, di_v, rows_v, b1_v,
                 sem_g, sem_s0, sem_s1, hist_sh, s1_sh, s2_sh):
  t = lax.axis_index("s")
  nb = t * NPT
  ebase = t * RPT

  pltpu.sync_copy(b1_hbm, b1_v)
  pltpu.sync_copy(src_hbm.at[pl.ds(ebase, RPT)], si_v)
  pltpu.sync_copy(dst_hbm.at[pl.ds(ebase, RPT)], di_v)

  z16 = jnp.zeros((16,), jnp.float32)

  def zero_deg(i, c):
    deg_v[pl.ds(i * 16, 16)] = z16
    return c

  lax.fori_loop(0, NP // 16, zero_deg, 0)

  ones = jnp.ones((16,), jnp.float32)

  def hist_row(r, c):
    for k in range(CW // 16):
      plsc.addupdate_scatter(deg_v, [di_v[r, pl.ds(k * 16, 16)]], ones)
    return c

  lax.fori_loop(0, RPT, hist_row, 0)
  pltpu.sync_copy(deg_v, hist_sh.at[t])

  _zero_rows(s_v, NPT)
  pltpu.sync_copy(s_v, s1_sh.at[pl.ds(nb, NPT)])
  pltpu.sync_copy(s_v, s2_sh.at[pl.ds(nb, NPT)])
  plsc.subcore_barrier()

  pltpu.sync_copy(hist_sh.at[:, pl.ds(nb, NPT)], tmp_v)

  def acc(i, c):
    sl = pl.ds(i * 16, 16)
    v = tmp_v[0, sl]
    for r in range(1, NT):
      v = v + tmp_v[r, sl]
    dinv_v[sl] = v
    return c

  lax.fori_loop(0, NPT // 16, acc, 0)

  def newton(i, c):
    d = dinv_v[pl.ds(i * 16, 16)] + 1.0
    bits = plsc.bitcast(d, jnp.int32)
    bits = jnp.int32(0x5F3759DF) - lax.shift_right_logical(bits, 1)
    y = plsc.bitcast(bits, jnp.float32)
    y = y * (1.5 - 0.5 * d * y * y)
    y = y * (1.5 - 0.5 * d * y * y)
    y = y * (1.5 - 0.5 * d * y * y)
    dinv_v[pl.ds(i * 16, 16)] = y
    return c

  lax.fori_loop(0, NPT // 16, newton, 0)

  def _splat(dvec, j):
    return dvec.at[jnp.full((16,), j, jnp.int32)].get(
        mode="promise_in_bounds")

  pltpu.sync_copy(h1_hbm.at[pl.ds(nb, NPT)], g_v)

  def scale(k, c):
    dvec = dinv_v[pl.ds(k * 16, 16)]
    for j in range(16):
      row = k * 16 + j
      g_v[row, :] = g_v[row, :] * _splat(dvec, j)
    return c

  lax.fori_loop(0, NPT // 16, scale, 0)
  pltpu.sync_copy(g_v, g1_hbm.at[pl.ds(nb, NPT)])
  plsc.subcore_barrier()

  def propagate(gtab_hbm, s_sh):
    sems = [sem_s0, sem_s1]

    def pair(p, c):
      for par in range(2):
        grp = p * 2 + par
        buf = rows_v.at[par]
        sem_s = sems[par]

        @pl.when(grp >= 2)
        def _drain():
          for j in range(GS):
            pltpu.make_async_copy(buf.at[j], s_sh.at[di_v.at[0]],
                                  sem_s).wait()

        gs = []
        for j in range(GS):
          row = grp * GS + j
          gs.append(pltpu.async_copy(gtab_hbm.at[si_v.at[row]], buf.at[j],
                                     sem_g))
        for gcopy in gs:
          gcopy.wait()
        for j in range(GS):
          row = grp * GS + j
          pltpu.async_copy(buf.at[j], s_sh.at[di_v.at[row]], sem_s, add=True)
      return c

    lax.fori_loop(0, NG // 2, pair, 0)
    for par in range(2):
      for j in range(GS):
        pltpu.make_async_copy(rows_v.at[par].at[j], s_sh.at[di_v.at[0]],
                              sems[par]).wait()

  propagate(g1_hbm, s1_sh)
  plsc.subcore_barrier()

  pltpu.sync_copy(s1_sh.at[pl.ds(nb, NPT)], s_v)
  b1v = b1_v[...]

  def mid(k, c):
    dvec = dinv_v[pl.ds(k * 16, 16)]
    for j in range(16):
      row = k * 16 + j
      dj = _splat(dvec, j)
      r = (s_v[row, :] + g_v[row, :]) * dj + b1v
      g_v[row, :] = jnp.maximum(r, 0.0) * dj
    return c

  lax.fori_loop(0, NPT // 16, mid, 0)
  pltpu.sync_copy(g_v, g2_hbm.at[pl.ds(nb, NPT)])
  plsc.subcore_barrier()

  propagate(g2_hbm, s2_sh)
  plsc.subcore_barrier()

  pltpu.sync_copy(s2_sh.at[pl.ds(nb, NPT)], s_v)

  def fin(k, c):
    dvec = dinv_v[pl.ds(k * 16, 16)]
    for j in range(16):
      row = k * 16 + j
      s_v[row, :] = (s_v[row, :] + g_v[row, :]) * _splat(dvec, j)
    return c

  lax.fori_loop(0, NPT // 16, fin, 0)
  pltpu.sync_copy(s_v, p2_hbm.at[pl.ds(nb, NPT)])


_BM = 1024


def _mm1_body(x_ref, w_ref, o_ref):
  o_ref[...] = jnp.dot(x_ref[...], w_ref[...],
                       preferred_element_type=jnp.float32)


def _mm2_body(p_ref, w_ref, b_ref, o_ref):
  o_ref[...] = jnp.dot(p_ref[...], w_ref[...],
                       preferred_element_type=jnp.float32) + b_ref[...]


def _mm1(xp, W1):
  return pl.pallas_call(
      _mm1_body,
      grid=(NP // _BM,),
      in_specs=[
          pl.BlockSpec((_BM, 128), lambda i: (i, 0)),
          pl.BlockSpec((128, D), lambda i: (0, 0)),
      ],
      out_specs=pl.BlockSpec((_BM, D), lambda i: (i, 0)),
      out_shape=jax.ShapeDtypeStruct((NP, D), jnp.float32),
  )(xp, W1)


_BM2 = 1000


def _mm2(p2, W2, b2):
  return pl.pallas_call(
      _mm2_body,
      grid=(N // _BM2,),
      in_specs=[
          pl.BlockSpec((_BM2, D), lambda i: (i, 0)),
          pl.BlockSpec((D, 128), lambda i: (0, 0)),
          pl.BlockSpec((1, 128), lambda i: (0, 0)),
      ],
      out_specs=pl.BlockSpec((_BM2, 128), lambda i: (i, 0)),
      out_shape=jax.ShapeDtypeStruct((N, 128), jnp.float32),
  )(p2, W2, b2)


@jax.jit
def kernel(x, edge_index, W1, b1, W2, b2):
  xp = jnp.pad(x, ((0, NP - N), (0, 0)))
  ep = jnp.pad(edge_index, ((0, 0), (0, EP - E)), constant_values=NP - 1)
  e3 = ep.reshape(2, ROWS, CW)
  h1 = _mm1(xp, W1)
  p2, _, _ = _prop_kernel(h1, e3[0], e3[1], b1)
  out = _mm2(p2[:N], W2, b2[None, :])
  return out


# speedup vs baseline: 1.3332x; 1.3332x over previous
import functools

import jax
import jax.numpy as jnp
from jax import lax
from jax.experimental import pallas as pl
from jax.experimental.pallas import tpu as pltpu
from jax.experimental.pallas import tpu_sc as plsc

N = 10000
E = 320000
D = 16
NT = 16
NP = 10240
NPT = NP // NT
CW = 128
ROWS = E // CW
RPT0 = 157
RPTB = 156
GS = 4
NG = RPTB // GS

_MESH = plsc.VectorSubcoreMesh(core_axis_name="c", subcore_axis_name="s",
                               num_cores=1)


def _zero_rows(ref, n):
  z = jnp.zeros((D,), jnp.float32)

  def body(i, c):
    ref[i, :] = z
    return c

  lax.fori_loop(0, n, body, 0)


@functools.partial(
    pl.kernel,
    out_type=(
        jax.ShapeDtypeStruct((NP, D), jnp.float32),
        jax.ShapeDtypeStruct((NP, D), jnp.float32),
        jax.ShapeDtypeStruct((NP, D), jnp.float32),
    ),
    mesh=_MESH,
    scratch_types=[
        pltpu.VMEM((NP,), jnp.float32),
        pltpu.VMEM((NT, NPT), jnp.float32),
        pltpu.VMEM((NPT,), jnp.float32),
        pltpu.VMEM((NPT, D), jnp.float32),
        pltpu.VMEM((NPT, D), jnp.float32),
        pltpu.VMEM((RPT0, CW), jnp.int32),
        pltpu.VMEM((RPT0, CW), jnp.int32),
        pltpu.VMEM((2, GS, CW, D), jnp.float32),
        pltpu.VMEM((D,), jnp.float32),
        pltpu.SemaphoreType.DMA,
        pltpu.SemaphoreType.DMA,
        pltpu.SemaphoreType.DMA,
        pltpu.VMEM_SHARED((NT, NP), jnp.float32),
        pltpu.VMEM_SHARED((NP, D), jnp.float32),
        pltpu.VMEM_SHARED((NP, D), jnp.float32),
    ],
    compiler_params=pltpu.CompilerParams(needs_layout_passes=False, use_tc_tiling_on_sc=False),
)
def _prop_kernel(h1_hbm, src_hbm, dst_hbm, b1_hbm,
                 p2_hbm, g1_hbm, g2_hbm,
                 deg_v, tmp_v, dinv_v, g_v, s_v, si_v, di_v, rows_v, b1_v,
                 sem_g, sem_s0, sem_s1, hist_sh, s1_sh, s2_sh):
  t = lax.axis_index("s")
  nb = t * NPT
  extra = (t < 4).astype(jnp.int32)
  nrows = RPTB + extra
  ebase = RPTB * t + jnp.minimum(t, 4)

  pltpu.sync_copy(b1_hbm, b1_v)

  @pl.when(t < 4)
  def _load_big():
    pltpu.sync_copy(src_hbm.at[pl.ds(ebase, RPT0)], si_v)
    pltpu.sync_copy(dst_hbm.at[pl.ds(ebase, RPT0)], di_v)

  @pl.when(t >= 4)
  def _load_small():
    pltpu.sync_copy(src_hbm.at[pl.ds(ebase, RPTB)], si_v.at[pl.ds(0, RPTB)])
    pltpu.sync_copy(dst_hbm.at[pl.ds(ebase, RPTB)], di_v.at[pl.ds(0, RPTB)])

  z16 = jnp.zeros((16,), jnp.float32)

  def zero_deg(i, c):
    deg_v[pl.ds(i * 16, 16)] = z16
    return c

  lax.fori_loop(0, NP // 16, zero_deg, 0)

  ones = jnp.ones((16,), jnp.float32)

  def hist_row(r, c):
    for k in range(CW // 16):
      plsc.addupdate_scatter(deg_v, [di_v[r, pl.ds(k * 16, 16)]], ones)
    return c

  lax.fori_loop(0, nrows, hist_row, 0)
  pltpu.sync_copy(deg_v, hist_sh.at[t])

  _zero_rows(s_v, NPT)
  pltpu.sync_copy(s_v, s1_sh.at[pl.ds(nb, NPT)])
  pltpu.sync_copy(s_v, s2_sh.at[pl.ds(nb, NPT)])
  plsc.subcore_barrier()

  pltpu.sync_copy(hist_sh.at[:, pl.ds(nb, NPT)], tmp_v)

  def acc(i, c):
    sl = pl.ds(i * 16, 16)
    v = tmp_v[0, sl]
    for r in range(1, NT):
      v = v + tmp_v[r, sl]
    dinv_v[sl] = v
    return c

  lax.fori_loop(0, NPT // 16, acc, 0)

  def newton(i, c):
    d = dinv_v[pl.ds(i * 16, 16)] + 1.0
    bits = plsc.bitcast(d, jnp.int32)
    bits = jnp.int32(0x5F3759DF) - lax.shift_right_logical(bits, 1)
    y = plsc.bitcast(bits, jnp.float32)
    y = y * (1.5 - 0.5 * d * y * y)
    y = y * (1.5 - 0.5 * d * y * y)
    y = y * (1.5 - 0.5 * d * y * y)
    dinv_v[pl.ds(i * 16, 16)] = y
    return c

  lax.fori_loop(0, NPT // 16, newton, 0)

  def _splat(dvec, j):
    return dvec.at[jnp.full((16,), j, jnp.int32)].get(
        mode="promise_in_bounds")

  pltpu.sync_copy(h1_hbm.at[pl.ds(nb, NPT)], g_v)

  def scale(k, c):
    dvec = dinv_v[pl.ds(k * 16, 16)]
    for j in range(16):
      row = k * 16 + j
      g_v[row, :] = g_v[row, :] * _splat(dvec, j)
    return c

  lax.fori_loop(0, NPT // 16, scale, 0)
  pltpu.sync_copy(g_v, g1_hbm.at[pl.ds(nb, NPT)])
  plsc.subcore_barrier()

  def propagate(gtab_hbm, s_sh):
    sems = [sem_s0, sem_s1]

    def pair(p, c):
      for par in range(2):
        grp = p * 2 + par
        buf = rows_v.at[par]
        sem_s = sems[par]

        @pl.when(grp >= 2)
        def _drain():
          for j in range(GS):
            pltpu.make_async_copy(buf.at[j], s_sh.at[di_v.at[0]],
                                  sem_s).wait()

        gs = []
        for j in range(GS):
          row = grp * GS + j
          gs.append(pltpu.async_copy(gtab_hbm.at[si_v.at[row]], buf.at[j],
                                     sem_g))
        for gcopy in gs:
          gcopy.wait()
        for j in range(GS):
          row = grp * GS + j
          pltpu.async_copy(buf.at[j], s_sh.at[di_v.at[row]], sem_s, add=True)
      return c

    lax.fori_loop(0, NG // 2, pair, 0)
    for j in range(GS):
      pltpu.make_async_copy(rows_v.at[0].at[j], s_sh.at[di_v.at[0]],
                            sem_s0).wait()
    gs = []
    for j in range(GS):
      row = (NG - 1) * GS + j
      gs.append(pltpu.async_copy(gtab_hbm.at[si_v.at[row]],
                                 rows_v.at[0].at[j], sem_g))
    for gcopy in gs:
      gcopy.wait()
    for j in range(GS):
      row = (NG - 1) * GS + j
      pltpu.async_copy(rows_v.at[0].at[j], s_sh.at[di_v.at[row]], sem_s0,
                       add=True)
    for j in range(GS):
      pltpu.make_async_copy(rows_v.at[1].at[j], s_sh.at[di_v.at[0]],
                            sem_s1).wait()

    @pl.when(t < 4)
    def _last_row():
      pltpu.sync_copy(gtab_hbm.at[si_v.at[RPTB]], rows_v.at[1].at[0])
      pltpu.sync_copy(rows_v.at[1].at[0], s_sh.at[di_v.at[RPTB]], add=True)

    for j in range(GS):
      pltpu.make_async_copy(rows_v.at[0].at[j], s_sh.at[di_v.at[0]],
                            sem_s0).wait()

  propagate(g1_hbm, s1_sh)
  plsc.subcore_barrier()

  pltpu.sync_copy(s1_sh.at[pl.ds(nb, NPT)], s_v)
  b1v = b1_v[...]

  def mid(k, c):
    dvec = dinv_v[pl.ds(k * 16, 16)]
    for j in range(16):
      row = k * 16 + j
      dj = _splat(dvec, j)
      r = (s_v[row, :] + g_v[row, :]) * dj + b1v
      g_v[row, :] = jnp.maximum(r, 0.0) * dj
    return c

  lax.fori_loop(0, NPT // 16, mid, 0)
  pltpu.sync_copy(g_v, g2_hbm.at[pl.ds(nb, NPT)])
  plsc.subcore_barrier()

  propagate(g2_hbm, s2_sh)
  plsc.subcore_barrier()

  pltpu.sync_copy(s2_sh.at[pl.ds(nb, NPT)], s_v)

  def fin(k, c):
    dvec = dinv_v[pl.ds(k * 16, 16)]
    for j in range(16):
      row = k * 16 + j
      s_v[row, :] = (s_v[row, :] + g_v[row, :]) * _splat(dvec, j)
    return c

  lax.fori_loop(0, NPT // 16, fin, 0)
  pltpu.sync_copy(s_v, p2_hbm.at[pl.ds(nb, NPT)])


_BM = 1024


def _mm1_body(x_ref, w_ref, o_ref):
  o_ref[...] = jnp.dot(x_ref[...], w_ref[...],
                       preferred_element_type=jnp.float32)


def _mm2_body(p_ref, w_ref, b_ref, o_ref):
  o_ref[...] = jnp.dot(p_ref[...], w_ref[...],
                       preferred_element_type=jnp.float32) + b_ref[...]


def _mm1(xp, W1):
  return pl.pallas_call(
      _mm1_body,
      grid=(NP // _BM,),
      in_specs=[
          pl.BlockSpec((_BM, 128), lambda i: (i, 0)),
          pl.BlockSpec((128, D), lambda i: (0, 0)),
      ],
      out_specs=pl.BlockSpec((_BM, D), lambda i: (i, 0)),
      out_shape=jax.ShapeDtypeStruct((NP, D), jnp.float32),
  )(xp, W1)


_BM2 = 1000


def _mm2(p2, W2, b2):
  return pl.pallas_call(
      _mm2_body,
      grid=(N // _BM2,),
      in_specs=[
          pl.BlockSpec((_BM2, D), lambda i: (i, 0)),
          pl.BlockSpec((D, 128), lambda i: (0, 0)),
          pl.BlockSpec((1, 128), lambda i: (0, 0)),
      ],
      out_specs=pl.BlockSpec((_BM2, 128), lambda i: (i, 0)),
      out_shape=jax.ShapeDtypeStruct((N, 128), jnp.float32),
  )(p2, W2, b2)


@jax.jit
def kernel(x, edge_index, W1, b1, W2, b2):
  xp = jnp.pad(x, ((0, NP - N), (0, 0)))
  e3 = edge_index.reshape(2, ROWS, CW)
  h1 = _mm1(xp, W1)
  p2, _, _ = _prop_kernel(h1, e3[0], e3[1], b1)
  out = _mm2(p2[:N], W2, b2[None, :])
  return out


# speedup vs baseline: 1.4879x; 1.1160x over previous
import functools

import jax
import jax.numpy as jnp
from jax import lax
from jax.experimental import pallas as pl
from jax.experimental.pallas import tpu as pltpu
from jax.experimental.pallas import tpu_sc as plsc

N = 10000
E = 320000
D = 16
NT = 16
NP = 10240
NPT = NP // NT
CW = 128
ROWS = E // CW
RPT0 = 157
RPTB = 156
GS = 6
NG = RPTB // GS

_MESH = plsc.VectorSubcoreMesh(core_axis_name="c", subcore_axis_name="s",
                               num_cores=1)


def _zero_rows(ref, n):
  z = jnp.zeros((D,), jnp.float32)

  def body(i, c):
    ref[i, :] = z
    return c

  lax.fori_loop(0, n, body, 0)


@functools.partial(
    pl.kernel,
    out_type=(
        jax.ShapeDtypeStruct((NP, D), jnp.float32),
        jax.ShapeDtypeStruct((NP, D), jnp.float32),
        jax.ShapeDtypeStruct((NP, D), jnp.float32),
    ),
    mesh=_MESH,
    scratch_types=[
        pltpu.VMEM((CW,), jnp.float32),
        pltpu.VMEM((NPT,), jnp.float32),
        pltpu.VMEM((NPT, D), jnp.float32),
        pltpu.VMEM((NPT, D), jnp.float32),
        pltpu.VMEM((RPT0, CW), jnp.int32),
        pltpu.VMEM((RPT0, CW), jnp.int32),
        pltpu.VMEM((2, GS, CW, D), jnp.float32),
        pltpu.VMEM((D,), jnp.float32),
        pltpu.SemaphoreType.DMA,
        pltpu.SemaphoreType.DMA,
        pltpu.SemaphoreType.DMA,
        pltpu.VMEM_SHARED((NP,), jnp.float32),
        pltpu.VMEM_SHARED((NP, D), jnp.float32),
        pltpu.VMEM_SHARED((NP, D), jnp.float32),
    ],
    compiler_params=pltpu.CompilerParams(needs_layout_passes=False, use_tc_tiling_on_sc=False),
)
def _prop_kernel(h1_hbm, src_hbm, dst_hbm, b1_hbm,
                 p2_hbm, g1_hbm, g2_hbm,
                 ones_v, dinv_v, g_v, s_v, si_v, di_v, rows_v, b1_v,
                 sem_g, sem_s0, sem_s1, deg_sh, s1_sh, s2_sh):
  t = lax.axis_index("s")
  nb = t * NPT
  extra = (t < 4).astype(jnp.int32)
  nrows = RPTB + extra
  ebase = RPTB * t + jnp.minimum(t, 4)

  pltpu.sync_copy(b1_hbm, b1_v)

  @pl.when(t < 4)
  def _load_big():
    pltpu.sync_copy(src_hbm.at[pl.ds(ebase, RPT0)], si_v)
    pltpu.sync_copy(dst_hbm.at[pl.ds(ebase, RPT0)], di_v)

  @pl.when(t >= 4)
  def _load_small():
    pltpu.sync_copy(src_hbm.at[pl.ds(ebase, RPTB)], si_v.at[pl.ds(0, RPTB)])
    pltpu.sync_copy(dst_hbm.at[pl.ds(ebase, RPTB)], di_v.at[pl.ds(0, RPTB)])

  z16 = jnp.zeros((16,), jnp.float32)

  def zero_ones(i, c):
    ones_v[pl.ds(i * 16, 16)] = z16 + 1.0
    dinv_v[pl.ds(i * 16, 16)] = z16
    return c

  lax.fori_loop(0, CW // 16, zero_ones, 0)

  def zero_dinv(i, c):
    dinv_v[pl.ds(i * 16, 16)] = z16
    return c

  lax.fori_loop(CW // 16, NPT // 16, zero_dinv, 0)
  pltpu.sync_copy(dinv_v, deg_sh.at[pl.ds(nb, NPT)])

  _zero_rows(s_v, NPT)
  pltpu.sync_copy(s_v, s1_sh.at[pl.ds(nb, NPT)])
  pltpu.sync_copy(s_v, s2_sh.at[pl.ds(nb, NPT)])
  plsc.subcore_barrier()

  def hist_fire(r, c):
    pltpu.async_copy(ones_v, deg_sh.at[di_v.at[r]], sem_s0, add=True)
    return c

  lax.fori_loop(0, nrows, hist_fire, 0)

  pltpu.sync_copy(h1_hbm.at[pl.ds(nb, NPT)], g_v)

  def hist_drain(r, c):
    pltpu.make_async_copy(ones_v, deg_sh.at[di_v.at[0]], sem_s0).wait()
    return c

  lax.fori_loop(0, nrows, hist_drain, 0)
  plsc.subcore_barrier()

  pltpu.sync_copy(deg_sh.at[pl.ds(nb, NPT)], dinv_v)

  def newton(i, c):
    d = dinv_v[pl.ds(i * 16, 16)] + 1.0
    bits = plsc.bitcast(d, jnp.int32)
    bits = jnp.int32(0x5F3759DF) - lax.shift_right_logical(bits, 1)
    y = plsc.bitcast(bits, jnp.float32)
    y = y * (1.5 - 0.5 * d * y * y)
    y = y * (1.5 - 0.5 * d * y * y)
    y = y * (1.5 - 0.5 * d * y * y)
    dinv_v[pl.ds(i * 16, 16)] = y
    return c

  lax.fori_loop(0, NPT // 16, newton, 0)

  def _splat(dvec, j):
    return dvec.at[jnp.full((16,), j, jnp.int32)].get(
        mode="promise_in_bounds")

  def scale(k, c):
    dvec = dinv_v[pl.ds(k * 16, 16)]
    for j in range(16):
      row = k * 16 + j
      g_v[row, :] = g_v[row, :] * _splat(dvec, j)
    return c

  lax.fori_loop(0, NPT // 16, scale, 0)
  pltpu.sync_copy(g_v, g1_hbm.at[pl.ds(nb, NPT)])
  plsc.subcore_barrier()

  def propagate(gtab_hbm, s_sh):
    sems = [sem_s0, sem_s1]

    def pair(p, c):
      for par in range(2):
        grp = p * 2 + par
        buf = rows_v.at[par]
        sem_s = sems[par]

        @pl.when(grp >= 2)
        def _drain():
          for j in range(GS):
            pltpu.make_async_copy(buf.at[j], s_sh.at[di_v.at[0]],
                                  sem_s).wait()

        gs = []
        for j in range(GS):
          row = grp * GS + j
          gs.append(pltpu.async_copy(gtab_hbm.at[si_v.at[row]], buf.at[j],
                                     sem_g))
        for gcopy in gs:
          gcopy.wait()
        for j in range(GS):
          row = grp * GS + j
          pltpu.async_copy(buf.at[j], s_sh.at[di_v.at[row]], sem_s, add=True)
      return c

    lax.fori_loop(0, NG // 2, pair, 0)
    for j in range(GS):
      pltpu.make_async_copy(rows_v.at[1].at[j], s_sh.at[di_v.at[0]],
                            sem_s1).wait()

    @pl.when(t < 4)
    def _last_row():
      pltpu.sync_copy(gtab_hbm.at[si_v.at[RPTB]], rows_v.at[1].at[0])
      pltpu.sync_copy(rows_v.at[1].at[0], s_sh.at[di_v.at[RPTB]], add=True)

    for j in range(GS):
      pltpu.make_async_copy(rows_v.at[0].at[j], s_sh.at[di_v.at[0]],
                            sem_s0).wait()

  propagate(g1_hbm, s1_sh)
  plsc.subcore_barrier()

  pltpu.sync_copy(s1_sh.at[pl.ds(nb, NPT)], s_v)
  b1v = b1_v[...]

  def mid(k, c):
    dvec = dinv_v[pl.ds(k * 16, 16)]
    for j in range(16):
      row = k * 16 + j
      dj = _splat(dvec, j)
      r = (s_v[row, :] + g_v[row, :]) * dj + b1v
      g_v[row, :] = jnp.maximum(r, 0.0) * dj
    return c

  lax.fori_loop(0, NPT // 16, mid, 0)
  pltpu.sync_copy(g_v, g2_hbm.at[pl.ds(nb, NPT)])
  plsc.subcore_barrier()

  propagate(g2_hbm, s2_sh)
  plsc.subcore_barrier()

  pltpu.sync_copy(s2_sh.at[pl.ds(nb, NPT)], s_v)

  def fin(k, c):
    dvec = dinv_v[pl.ds(k * 16, 16)]
    for j in range(16):
      row = k * 16 + j
      s_v[row, :] = (s_v[row, :] + g_v[row, :]) * _splat(dvec, j)
    return c

  lax.fori_loop(0, NPT // 16, fin, 0)
  pltpu.sync_copy(s_v, p2_hbm.at[pl.ds(nb, NPT)])


_BM = 1024


def _mm1_body(x_ref, w_ref, o_ref):
  o_ref[...] = jnp.dot(x_ref[...], w_ref[...],
                       preferred_element_type=jnp.float32)


def _mm2_body(p_ref, w_ref, b_ref, o_ref):
  o_ref[...] = jnp.dot(p_ref[...], w_ref[...],
                       preferred_element_type=jnp.float32) + b_ref[...]


def _mm1(xp, W1):
  return pl.pallas_call(
      _mm1_body,
      grid=(NP // _BM,),
      in_specs=[
          pl.BlockSpec((_BM, 128), lambda i: (i, 0)),
          pl.BlockSpec((128, D), lambda i: (0, 0)),
      ],
      out_specs=pl.BlockSpec((_BM, D), lambda i: (i, 0)),
      out_shape=jax.ShapeDtypeStruct((NP, D), jnp.float32),
  )(xp, W1)


_BM2 = 1000


def _mm2(p2, W2, b2):
  return pl.pallas_call(
      _mm2_body,
      grid=(N // _BM2,),
      in_specs=[
          pl.BlockSpec((_BM2, D), lambda i: (i, 0)),
          pl.BlockSpec((D, 128), lambda i: (0, 0)),
          pl.BlockSpec((1, 128), lambda i: (0, 0)),
      ],
      out_specs=pl.BlockSpec((_BM2, 128), lambda i: (i, 0)),
      out_shape=jax.ShapeDtypeStruct((N, 128), jnp.float32),
  )(p2, W2, b2)


@jax.jit
def kernel(x, edge_index, W1, b1, W2, b2):
  xp = jnp.pad(x, ((0, NP - N), (0, 0)))
  e3 = edge_index.reshape(2, ROWS, CW)
  h1 = _mm1(xp, W1)
  p2, _, _ = _prop_kernel(h1, e3[0], e3[1], b1)
  out = _mm2(p2[:N], W2, b2[None, :])
  return out


# speedup vs baseline: 1.7162x; 1.1535x over previous
import functools

import jax
import jax.numpy as jnp
from jax import lax
from jax.experimental import pallas as pl
from jax.experimental.pallas import tpu as pltpu
from jax.experimental.pallas import tpu_sc as plsc

N = 10000
E = 320000
D = 16
NT = 16
NP = N
NPT = 640
CW = 128
ROWS = E // CW
RPT0 = 157
RPTB = 156
GS = 6
NG = RPTB // GS

_MESH = plsc.VectorSubcoreMesh(core_axis_name="c", subcore_axis_name="s",
                               num_cores=1)


def _zero_rows(ref, n):
  z = jnp.zeros((D,), jnp.float32)

  def body(i, c):
    ref[i, :] = z
    return c

  lax.fori_loop(0, n, body, 0)


@functools.partial(
    pl.kernel,
    out_type=(
        jax.ShapeDtypeStruct((NP, D), jnp.float32),
        jax.ShapeDtypeStruct((NP, D), jnp.float32),
        jax.ShapeDtypeStruct((NP, D), jnp.float32),
    ),
    mesh=_MESH,
    scratch_types=[
        pltpu.VMEM((CW,), jnp.float32),
        pltpu.VMEM((NPT,), jnp.float32),
        pltpu.VMEM((NPT, D), jnp.float32),
        pltpu.VMEM((NPT, D), jnp.float32),
        pltpu.VMEM((RPT0, CW), jnp.int32),
        pltpu.VMEM((RPT0, CW), jnp.int32),
        pltpu.VMEM((2, GS, CW, D), jnp.float32),
        pltpu.VMEM((D,), jnp.float32),
        pltpu.SemaphoreType.DMA,
        pltpu.SemaphoreType.DMA,
        pltpu.SemaphoreType.DMA,
        pltpu.VMEM_SHARED((NP,), jnp.float32),
        pltpu.VMEM_SHARED((NP, D), jnp.float32),
        pltpu.VMEM_SHARED((NP, D), jnp.float32),
    ],
    compiler_params=pltpu.CompilerParams(needs_layout_passes=False, use_tc_tiling_on_sc=False),
)
def _prop_kernel(h1_hbm, e3_hbm, b1_hbm,
                 p2_hbm, g1_hbm, g2_hbm,
                 ones_v, dinv_v, g_v, s_v, si_v, di_v, rows_v, b1_v,
                 sem_g, sem_s0, sem_s1, deg_sh, s1_sh, s2_sh):
  src_hbm = e3_hbm.at[0]
  dst_hbm = e3_hbm.at[1]
  t = lax.axis_index("s")
  nb = jnp.minimum(t * NPT, N - NPT)
  extra = (t < 4).astype(jnp.int32)
  nrows = RPTB + extra
  ebase = RPTB * t + jnp.minimum(t, 4)

  pltpu.sync_copy(b1_hbm, b1_v)

  @pl.when(t < 4)
  def _load_big():
    pltpu.sync_copy(src_hbm.at[pl.ds(ebase, RPT0)], si_v)
    pltpu.sync_copy(dst_hbm.at[pl.ds(ebase, RPT0)], di_v)

  @pl.when(t >= 4)
  def _load_small():
    pltpu.sync_copy(src_hbm.at[pl.ds(ebase, RPTB)], si_v.at[pl.ds(0, RPTB)])
    pltpu.sync_copy(dst_hbm.at[pl.ds(ebase, RPTB)], di_v.at[pl.ds(0, RPTB)])

  z16 = jnp.zeros((16,), jnp.float32)

  def zero_ones(i, c):
    ones_v[pl.ds(i * 16, 16)] = z16 + 1.0
    dinv_v[pl.ds(i * 16, 16)] = z16
    return c

  lax.fori_loop(0, CW // 16, zero_ones, 0)

  def zero_dinv(i, c):
    dinv_v[pl.ds(i * 16, 16)] = z16
    return c

  lax.fori_loop(CW // 16, NPT // 16, zero_dinv, 0)
  pltpu.sync_copy(dinv_v, deg_sh.at[pl.ds(nb, NPT)])

  _zero_rows(s_v, NPT)
  pltpu.sync_copy(s_v, s1_sh.at[pl.ds(nb, NPT)])
  pltpu.sync_copy(s_v, s2_sh.at[pl.ds(nb, NPT)])
  plsc.subcore_barrier()

  def hist_fire(r, c):
    pltpu.async_copy(ones_v, deg_sh.at[di_v.at[r]], sem_s0, add=True)
    return c

  lax.fori_loop(0, nrows, hist_fire, 0)

  pltpu.sync_copy(h1_hbm.at[pl.ds(nb, NPT)], g_v)

  def hist_drain(r, c):
    pltpu.make_async_copy(ones_v, deg_sh.at[di_v.at[0]], sem_s0).wait()
    return c

  lax.fori_loop(0, nrows, hist_drain, 0)
  plsc.subcore_barrier()

  pltpu.sync_copy(deg_sh.at[pl.ds(nb, NPT)], dinv_v)

  def newton(i, c):
    d = dinv_v[pl.ds(i * 16, 16)] + 1.0
    bits = plsc.bitcast(d, jnp.int32)
    bits = jnp.int32(0x5F3759DF) - lax.shift_right_logical(bits, 1)
    y = plsc.bitcast(bits, jnp.float32)
    y = y * (1.5 - 0.5 * d * y * y)
    y = y * (1.5 - 0.5 * d * y * y)
    y = y * (1.5 - 0.5 * d * y * y)
    dinv_v[pl.ds(i * 16, 16)] = y
    return c

  lax.fori_loop(0, NPT // 16, newton, 0)

  def _splat(dvec, j):
    return dvec.at[jnp.full((16,), j, jnp.int32)].get(
        mode="promise_in_bounds")

  def scale(k, c):
    dvec = dinv_v[pl.ds(k * 16, 16)]
    for j in range(16):
      row = k * 16 + j
      g_v[row, :] = g_v[row, :] * _splat(dvec, j)
    return c

  lax.fori_loop(0, NPT // 16, scale, 0)
  pltpu.sync_copy(g_v, g1_hbm.at[pl.ds(nb, NPT)])
  plsc.subcore_barrier()

  def propagate(gtab_hbm, s_sh):
    sems = [sem_s0, sem_s1]

    def pair(p, c):
      for par in range(2):
        grp = p * 2 + par
        buf = rows_v.at[par]
        sem_s = sems[par]

        @pl.when(grp >= 2)
        def _drain():
          for j in range(GS):
            pltpu.make_async_copy(buf.at[j], s_sh.at[di_v.at[0]],
                                  sem_s).wait()

        gs = []
        for j in range(GS):
          row = grp * GS + j
          gs.append(pltpu.async_copy(gtab_hbm.at[si_v.at[row]], buf.at[j],
                                     sem_g))
        for gcopy in gs:
          gcopy.wait()
        for j in range(GS):
          row = grp * GS + j
          pltpu.async_copy(buf.at[j], s_sh.at[di_v.at[row]], sem_s, add=True)
      return c

    lax.fori_loop(0, NG // 2, pair, 0)
    for j in range(GS):
      pltpu.make_async_copy(rows_v.at[1].at[j], s_sh.at[di_v.at[0]],
                            sem_s1).wait()

    @pl.when(t < 4)
    def _last_row():
      pltpu.sync_copy(gtab_hbm.at[si_v.at[RPTB]], rows_v.at[1].at[0])
      pltpu.sync_copy(rows_v.at[1].at[0], s_sh.at[di_v.at[RPTB]], add=True)

    for j in range(GS):
      pltpu.make_async_copy(rows_v.at[0].at[j], s_sh.at[di_v.at[0]],
                            sem_s0).wait()

  propagate(g1_hbm, s1_sh)
  plsc.subcore_barrier()

  pltpu.sync_copy(s1_sh.at[pl.ds(nb, NPT)], s_v)
  b1v = b1_v[...]

  def mid(k, c):
    dvec = dinv_v[pl.ds(k * 16, 16)]
    for j in range(16):
      row = k * 16 + j
      dj = _splat(dvec, j)
      r = (s_v[row, :] + g_v[row, :]) * dj + b1v
      g_v[row, :] = jnp.maximum(r, 0.0) * dj
    return c

  lax.fori_loop(0, NPT // 16, mid, 0)
  pltpu.sync_copy(g_v, g2_hbm.at[pl.ds(nb, NPT)])
  plsc.subcore_barrier()

  propagate(g2_hbm, s2_sh)
  plsc.subcore_barrier()

  pltpu.sync_copy(s2_sh.at[pl.ds(nb, NPT)], s_v)

  def fin(k, c):
    dvec = dinv_v[pl.ds(k * 16, 16)]
    for j in range(16):
      row = k * 16 + j
      s_v[row, :] = (s_v[row, :] + g_v[row, :]) * _splat(dvec, j)
    return c

  lax.fori_loop(0, NPT // 16, fin, 0)
  pltpu.sync_copy(s_v, p2_hbm.at[pl.ds(nb, NPT)])


_BM = 2000


def _mm1_body(x_ref, w_ref, o_ref):
  o_ref[...] = jnp.dot(x_ref[...], w_ref[...],
                       preferred_element_type=jnp.float32)


def _mm2_body(p_ref, w_ref, b_ref, o_ref):
  o_ref[...] = jnp.dot(p_ref[...], w_ref[...],
                       preferred_element_type=jnp.float32) + b_ref[...]


def _mm1(x, W1):
  return pl.pallas_call(
      _mm1_body,
      grid=(N // _BM,),
      in_specs=[
          pl.BlockSpec((_BM, 128), lambda i: (i, 0)),
          pl.BlockSpec((128, D), lambda i: (0, 0)),
      ],
      out_specs=pl.BlockSpec((_BM, D), lambda i: (i, 0)),
      out_shape=jax.ShapeDtypeStruct((N, D), jnp.float32),
  )(x, W1)


_BM2 = 2000


def _mm2(p2, W2, b2):
  return pl.pallas_call(
      _mm2_body,
      grid=(N // _BM2,),
      in_specs=[
          pl.BlockSpec((_BM2, D), lambda i: (i, 0)),
          pl.BlockSpec((D, 128), lambda i: (0, 0)),
          pl.BlockSpec((1, 128), lambda i: (0, 0)),
      ],
      out_specs=pl.BlockSpec((_BM2, 128), lambda i: (i, 0)),
      out_shape=jax.ShapeDtypeStruct((N, 128), jnp.float32),
  )(p2, W2, b2)


@jax.jit
def kernel(x, edge_index, W1, b1, W2, b2):
  e3 = edge_index.reshape(2, ROWS, CW)
  h1 = _mm1(x, W1)
  p2, _, _ = _prop_kernel(h1, e3, b1)
  out = _mm2(p2, W2, b2[None, :])
  return out


# speedup vs baseline: 2.1125x; 1.2309x over previous
import functools

import jax
import jax.numpy as jnp
from jax import lax
from jax.experimental import pallas as pl
from jax.experimental.pallas import tpu as pltpu
from jax.experimental.pallas import tpu_sc as plsc

N = 10000
E = 320000
D = 16
NT = 16
NP = N
NPT = 640
CW = 128
ROWS = E // CW
RPT0 = 157
RPTB = 156
GS = 6
NG = RPTB // GS

_MESH = plsc.VectorSubcoreMesh(core_axis_name="c", subcore_axis_name="s",
                               num_cores=1)


def _zero_rows(ref, n):
  z = jnp.zeros((D,), jnp.float32)

  def body(i, c):
    ref[i, :] = z
    return c

  lax.fori_loop(0, n, body, 0)


@functools.partial(
    pl.kernel,
    out_type=jax.ShapeDtypeStruct((NP, D), jnp.float32),
    mesh=_MESH,
    scratch_types=[
        pltpu.VMEM((CW,), jnp.float32),
        pltpu.VMEM((NPT,), jnp.float32),
        pltpu.VMEM((NPT, D), jnp.float32),
        pltpu.VMEM((NPT, D), jnp.float32),
        pltpu.VMEM((RPT0, CW), jnp.int32),
        pltpu.VMEM((RPT0, CW), jnp.int32),
        pltpu.VMEM((2, GS, CW, D), jnp.float32),
        pltpu.VMEM((D,), jnp.float32),
        pltpu.SemaphoreType.DMA,
        pltpu.SemaphoreType.DMA,
        pltpu.SemaphoreType.DMA,
        pltpu.VMEM_SHARED((NP,), jnp.float32),
        pltpu.VMEM_SHARED((NP, D), jnp.float32),
        pltpu.VMEM_SHARED((NP, D), jnp.float32),
        pltpu.VMEM_SHARED((NP, D), jnp.float32),
    ],
    compiler_params=pltpu.CompilerParams(needs_layout_passes=False, use_tc_tiling_on_sc=False),
)
def _prop_kernel(h1_hbm, e3_hbm, b1_hbm,
                 p2_hbm,
                 ones_v, dinv_v, g_v, s_v, si_v, di_v, rows_v, b1_v,
                 sem_g, sem_s0, sem_s1, deg_sh, g_sh, s1_sh, s2_sh):
  src_hbm = e3_hbm.at[0]
  dst_hbm = e3_hbm.at[1]
  t = lax.axis_index("s")
  nb = jnp.minimum(t * NPT, N - NPT)
  extra = (t < 4).astype(jnp.int32)
  nrows = RPTB + extra
  ebase = RPTB * t + jnp.minimum(t, 4)

  pltpu.sync_copy(b1_hbm, b1_v)

  @pl.when(t < 4)
  def _load_big():
    pltpu.sync_copy(src_hbm.at[pl.ds(ebase, RPT0)], si_v)
    pltpu.sync_copy(dst_hbm.at[pl.ds(ebase, RPT0)], di_v)

  @pl.when(t >= 4)
  def _load_small():
    pltpu.sync_copy(src_hbm.at[pl.ds(ebase, RPTB)], si_v.at[pl.ds(0, RPTB)])
    pltpu.sync_copy(dst_hbm.at[pl.ds(ebase, RPTB)], di_v.at[pl.ds(0, RPTB)])

  z16 = jnp.zeros((16,), jnp.float32)

  def zero_ones(i, c):
    ones_v[pl.ds(i * 16, 16)] = z16 + 1.0
    dinv_v[pl.ds(i * 16, 16)] = z16
    return c

  lax.fori_loop(0, CW // 16, zero_ones, 0)

  def zero_dinv(i, c):
    dinv_v[pl.ds(i * 16, 16)] = z16
    return c

  lax.fori_loop(CW // 16, NPT // 16, zero_dinv, 0)
  pltpu.sync_copy(dinv_v, deg_sh.at[pl.ds(nb, NPT)])

  _zero_rows(s_v, NPT)
  pltpu.sync_copy(s_v, s1_sh.at[pl.ds(nb, NPT)])
  pltpu.sync_copy(s_v, s2_sh.at[pl.ds(nb, NPT)])
  plsc.subcore_barrier()

  def hist_fire(r, c):
    pltpu.async_copy(ones_v, deg_sh.at[di_v.at[r]], sem_s0, add=True)
    return c

  lax.fori_loop(0, nrows, hist_fire, 0)

  pltpu.sync_copy(h1_hbm.at[pl.ds(nb, NPT)], g_v)

  def hist_drain(r, c):
    pltpu.make_async_copy(ones_v, deg_sh.at[di_v.at[0]], sem_s0).wait()
    return c

  lax.fori_loop(0, nrows, hist_drain, 0)
  plsc.subcore_barrier()

  pltpu.sync_copy(deg_sh.at[pl.ds(nb, NPT)], dinv_v)

  def newton(i, c):
    d = dinv_v[pl.ds(i * 16, 16)] + 1.0
    bits = plsc.bitcast(d, jnp.int32)
    bits = jnp.int32(0x5F3759DF) - lax.shift_right_logical(bits, 1)
    y = plsc.bitcast(bits, jnp.float32)
    y = y * (1.5 - 0.5 * d * y * y)
    y = y * (1.5 - 0.5 * d * y * y)
    y = y * (1.5 - 0.5 * d * y * y)
    dinv_v[pl.ds(i * 16, 16)] = y
    return c

  lax.fori_loop(0, NPT // 16, newton, 0)

  def _splat(dvec, j):
    return dvec.at[jnp.full((16,), j, jnp.int32)].get(
        mode="promise_in_bounds")

  def scale(k, c):
    dvec = dinv_v[pl.ds(k * 16, 16)]
    for j in range(16):
      row = k * 16 + j
      g_v[row, :] = g_v[row, :] * _splat(dvec, j)
    return c

  lax.fori_loop(0, NPT // 16, scale, 0)
  pltpu.sync_copy(g_v, g_sh.at[pl.ds(nb, NPT)])
  plsc.subcore_barrier()

  def propagate(s_sh):
    sems = [sem_s0, sem_s1]

    def pair(p, c):
      for par in range(2):
        grp = p * 2 + par
        buf = rows_v.at[par]
        sem_s = sems[par]

        @pl.when(grp >= 2)
        def _drain():
          for j in range(GS):
            pltpu.make_async_copy(buf.at[j], s_sh.at[di_v.at[0]],
                                  sem_s).wait()

        gs = []
        for j in range(GS):
          row = grp * GS + j
          gs.append(pltpu.async_copy(g_sh.at[si_v.at[row]], buf.at[j],
                                     sem_g))
        for j in range(GS):
          row = grp * GS + j
          gs[j].wait()
          pltpu.async_copy(buf.at[j], s_sh.at[di_v.at[row]], sem_s, add=True)
      return c

    lax.fori_loop(0, NG // 2, pair, 0)
    for j in range(GS):
      pltpu.make_async_copy(rows_v.at[1].at[j], s_sh.at[di_v.at[0]],
                            sem_s1).wait()

    @pl.when(t < 4)
    def _last_row():
      pltpu.sync_copy(g_sh.at[si_v.at[RPTB]], rows_v.at[1].at[0])
      pltpu.sync_copy(rows_v.at[1].at[0], s_sh.at[di_v.at[RPTB]], add=True)

    for j in range(GS):
      pltpu.make_async_copy(rows_v.at[0].at[j], s_sh.at[di_v.at[0]],
                            sem_s0).wait()

  propagate(s1_sh)
  plsc.subcore_barrier()

  pltpu.sync_copy(s1_sh.at[pl.ds(nb, NPT)], s_v)
  b1v = b1_v[...]

  def mid(k, c):
    dvec = dinv_v[pl.ds(k * 16, 16)]
    for j in range(16):
      row = k * 16 + j
      dj = _splat(dvec, j)
      r = (s_v[row, :] + g_v[row, :]) * dj + b1v
      g_v[row, :] = jnp.maximum(r, 0.0) * dj
    return c

  lax.fori_loop(0, NPT // 16, mid, 0)
  pltpu.sync_copy(g_v, g_sh.at[pl.ds(nb, NPT)])
  plsc.subcore_barrier()

  propagate(s2_sh)
  plsc.subcore_barrier()

  pltpu.sync_copy(s2_sh.at[pl.ds(nb, NPT)], s_v)

  def fin(k, c):
    dvec = dinv_v[pl.ds(k * 16, 16)]
    for j in range(16):
      row = k * 16 + j
      s_v[row, :] = (s_v[row, :] + g_v[row, :]) * _splat(dvec, j)
    return c

  lax.fori_loop(0, NPT // 16, fin, 0)
  pltpu.sync_copy(s_v, p2_hbm.at[pl.ds(nb, NPT)])


_BM = 2000


def _mm1_body(x_ref, w_ref, o_ref):
  o_ref[...] = jnp.dot(x_ref[...], w_ref[...],
                       preferred_element_type=jnp.float32)


def _mm2_body(p_ref, w_ref, b_ref, o_ref):
  o_ref[...] = jnp.dot(p_ref[...], w_ref[...],
                       preferred_element_type=jnp.float32) + b_ref[...]


def _mm1(x, W1):
  return pl.pallas_call(
      _mm1_body,
      grid=(N // _BM,),
      in_specs=[
          pl.BlockSpec((_BM, 128), lambda i: (i, 0)),
          pl.BlockSpec((128, D), lambda i: (0, 0)),
      ],
      out_specs=pl.BlockSpec((_BM, D), lambda i: (i, 0)),
      out_shape=jax.ShapeDtypeStruct((N, D), jnp.float32),
  )(x, W1)


_BM2 = 2000


def _mm2(p2, W2, b2):
  return pl.pallas_call(
      _mm2_body,
      grid=(N // _BM2,),
      in_specs=[
          pl.BlockSpec((_BM2, D), lambda i: (i, 0)),
          pl.BlockSpec((D, 128), lambda i: (0, 0)),
          pl.BlockSpec((1, 128), lambda i: (0, 0)),
      ],
      out_specs=pl.BlockSpec((_BM2, 128), lambda i: (i, 0)),
      out_shape=jax.ShapeDtypeStruct((N, 128), jnp.float32),
  )(p2, W2, b2)


@jax.jit
def kernel(x, edge_index, W1, b1, W2, b2):
  e3 = edge_index.reshape(2, ROWS, CW)
  h1 = _mm1(x, W1)
  p2 = _prop_kernel(h1, e3, b1)
  out = _mm2(p2, W2, b2[None, :])
  return out
